# baseline (speedup 1.0000x reference)
import jax
import jax.numpy as jnp
from jax.experimental import pallas as pl
from jax.experimental.pallas import tpu as pltpu


def _fold_kernel(wqkv_ref, weff_ref):
    w = wqkv_ref[0]
    d = w.shape[0]
    weff_ref[0] = (w[:, :d] + w[:, d:2 * d] + w[:, 2 * d:]).astype(jnp.bfloat16)


def _main_kernel(x_ref, wg_ref, bg_ref, weff_ref, wproj_ref, bproj_ref,
                 o_ref, s_scr):
    blk, dim = x_ref.shape
    n_exp = wg_ref.shape[1]
    heads = 4
    dh = dim // heads
    scale = dh ** -0.5
    par = jax.lax.rem(pl.program_id(0), 2)

    xb = x_ref[...]
    scores = jnp.dot(xb, wg_ref[...], preferred_element_type=jnp.float32)
    scores = scores + bg_ref[...]
    m = jnp.max(scores, axis=1, keepdims=True)
    ex = jnp.exp(scores - m)
    p = ex / jnp.sum(ex, axis=1, keepdims=True)

    lane = jax.lax.broadcasted_iota(jnp.int32, p.shape, 1)
    i1 = jnp.argmax(p, axis=1)[:, None]
    m1 = lane == i1
    v1 = jnp.max(p, axis=1, keepdims=True)
    p2 = jnp.where(m1, -1e30, p)
    i2 = jnp.argmax(p2, axis=1)[:, None]
    m2 = lane == i2
    v2 = jnp.max(p2, axis=1, keepdims=True)
    w = jnp.where(m1, v1, 0.0) + jnp.where(m2, v2, 0.0)

    xb16 = xb.astype(jnp.bfloat16)
    s = jnp.zeros((blk, dim), jnp.float32)
    for e in range(n_exp):
        se = jnp.dot(xb16, weff_ref[e], preferred_element_type=jnp.float32)
        s = s + w[:, e:e + 1] * se
    s_scr[pl.ds(par, 1), :, :] = s[None]

    sp = s_scr[pl.ds(1 - par, 1), :, :][0]
    sh = [sp[:, j * dh:(j + 1) * dh] for j in range(heads)]
    outs = []
    for i in range(heads):
        lg = jnp.concatenate(
            [jnp.sum(sh[i] * sh[j], axis=1, keepdims=True) * scale
             for j in range(heads)], axis=1)
        mx = jnp.max(lg, axis=1, keepdims=True)
        el = jnp.exp(lg - mx)
        pr = el / jnp.sum(el, axis=1, keepdims=True)
        acc = pr[:, 0:1] * sh[0]
        for j in range(1, heads):
            acc = acc + pr[:, j:j + 1] * sh[j]
        outs.append(acc)
    attn_out = jnp.concatenate(outs, axis=1)

    o_ref[...] = (jnp.dot(attn_out.astype(jnp.bfloat16), wproj_ref[...],
                          preferred_element_type=jnp.float32)
                  + bproj_ref[...])


def kernel(x, Wg, bg, Wqkv, Wproj, bproj):
    b, dim = x.shape
    n_exp = Wg.shape[1]
    heads = 4
    dh = dim // heads
    blk = 512
    nblk = b // blk

    wproj_perm = Wproj.reshape(dh, heads, dim).swapaxes(0, 1).reshape(dim, dim)

    weff = pl.pallas_call(
        _fold_kernel,
        grid=(n_exp,),
        in_specs=[pl.BlockSpec((1, dim, 3 * dim), lambda e: (e, 0, 0))],
        out_specs=pl.BlockSpec((1, dim, dim), lambda e: (e, 0, 0)),
        out_shape=jax.ShapeDtypeStruct((n_exp, dim, dim), jnp.bfloat16),
    )(Wqkv)

    out = pl.pallas_call(
        _main_kernel,
        grid=(nblk + 1,),
        in_specs=[
            pl.BlockSpec((blk, dim), lambda i: (jnp.minimum(i, nblk - 1), 0)),
            pl.BlockSpec((dim, n_exp), lambda i: (0, 0)),
            pl.BlockSpec((1, n_exp), lambda i: (0, 0)),
            pl.BlockSpec((n_exp, dim, dim), lambda i: (0, 0, 0)),
            pl.BlockSpec((dim, dim), lambda i: (0, 0)),
            pl.BlockSpec((1, dim), lambda i: (0, 0)),
        ],
        out_specs=pl.BlockSpec((blk, dim),
                               lambda i: (jnp.maximum(i - 1, 0), 0)),
        out_shape=jax.ShapeDtypeStruct((b, dim), jnp.float32),
        scratch_shapes=[pltpu.VMEM((2, blk, dim), jnp.float32)],
    )(x, Wg, bg.reshape(1, n_exp), weff, wproj_perm.astype(jnp.bfloat16),
      bproj.reshape(1, dim))
    return out


# speedup vs baseline: 1.3971x; 1.3971x over previous
import jax
import jax.numpy as jnp
import numpy as np
from jax.experimental import pallas as pl

_HEADS = 4
_DHP = 256


def _fold_kernel(wqkv_ref, weff_ref):
    w = wqkv_ref[0]
    d = w.shape[0]
    dh = d // _HEADS
    weff = (w[:, :d] + w[:, d:2 * d] + w[:, 2 * d:]).astype(jnp.bfloat16)
    if _DHP == dh:
        weff_ref[...] = weff
    else:
        z = jnp.zeros((d, _DHP - dh), jnp.bfloat16)
        weff_ref[...] = jnp.concatenate(
            [jnp.concatenate([weff[:, h * dh:(h + 1) * dh], z], axis=1)
             for h in range(_HEADS)], axis=1)


def _main_kernel(x_ref, wg_ref, bg_ref, weff_ref, wproj_ref, bproj_ref,
                 bd_ref, kb2_ref, o_ref):
    blk, dim = x_ref.shape
    n_exp = wg_ref.shape[1]
    heads = _HEADS
    dh = dim // heads
    dhp = _DHP
    dimp = heads * dhp
    scale = dh ** -0.5
    xb = x_ref[...]
    scores = jnp.dot(xb, wg_ref[...], preferred_element_type=jnp.float32)
    scores = scores + bg_ref[...]
    m = jnp.max(scores, axis=1, keepdims=True)
    ex = jnp.exp(scores - m)
    p = ex / jnp.sum(ex, axis=1, keepdims=True)
    lane = jax.lax.broadcasted_iota(jnp.int32, p.shape, 1)
    i1 = jnp.argmax(p, axis=1)[:, None]
    m1 = lane == i1
    v1 = jnp.max(p, axis=1, keepdims=True)
    p2 = jnp.where(m1, -1e30, p)
    i2 = jnp.argmax(p2, axis=1)[:, None]
    m2 = lane == i2
    v2 = jnp.max(p2, axis=1, keepdims=True)
    w = jnp.where(m1, v1, 0.0) + jnp.where(m2, v2, 0.0)

    w16 = w.astype(jnp.bfloat16)
    xb16 = xb.astype(jnp.bfloat16)
    xs = jnp.concatenate(
        [xb16 * w16[:, e:e + 1] for e in range(n_exp)],
        axis=1)
    s = jnp.dot(xs, weff_ref[...],
                preferred_element_type=jnp.float32)

    s16 = s.astype(jnp.bfloat16)
    sh16 = [s16[:, i * dhp:(i + 1) * dhp] for i in range(heads)]
    pr_chunks = []
    for i in range(heads):
        prods = jnp.concatenate([sh16[i] * sh16[j] for j in range(heads)],
                                axis=1)
        g = jnp.dot(prods, bd_ref[...],
                    preferred_element_type=jnp.float32) * scale
        eg = jnp.exp(g)
        pr_chunks.append(eg / jnp.sum(eg, axis=1, keepdims=True))
    attn_parts = []
    for i in range(heads):
        prb = jnp.dot(pr_chunks[i].astype(jnp.bfloat16), kb2_ref[...],
                      preferred_element_type=jnp.float32)
        ob = prb * s
        acc = ob[:, 0:dhp]
        for j in range(1, heads):
            acc = acc + ob[:, j * dhp:(j + 1) * dhp]
        attn_parts.append(acc)
    attn_out = jnp.concatenate(attn_parts, axis=1)

    o_ref[...] = (jnp.dot(attn_out.astype(jnp.bfloat16), wproj_ref[...],
                          preferred_element_type=jnp.float32)
                  + bproj_ref[...])


def kernel(x, Wg, bg, Wqkv, Wproj, bproj):
    b, dim = x.shape
    n_exp = Wg.shape[1]
    heads = _HEADS
    dh = dim // heads
    dhp = _DHP
    dimp = heads * dhp
    blk = 512
    nblk = b // blk

    wproj_perm = Wproj.reshape(dh, heads, dim).swapaxes(0, 1)
    wproj_pad = jnp.concatenate(
        [wproj_perm, jnp.zeros((heads, dhp - dh, dim), Wproj.dtype)],
        axis=1).reshape(dimp, dim)

    bd = jnp.asarray(np.kron(np.eye(heads), np.ones((dhp, 1))),
                     jnp.bfloat16)
    kb2 = jnp.asarray(np.kron(np.eye(heads), np.ones((1, dhp))),
                      jnp.bfloat16)

    weff = pl.pallas_call(
        _fold_kernel,
        grid=(n_exp,),
        in_specs=[pl.BlockSpec((1, dim, 3 * dim), lambda e: (e, 0, 0))],
        out_specs=pl.BlockSpec((dim, dimp), lambda e: (e, 0)),
        out_shape=jax.ShapeDtypeStruct((n_exp * dim, dimp), jnp.bfloat16),
    )(Wqkv)

    out = pl.pallas_call(
        _main_kernel,
        grid=(nblk,),
        in_specs=[
            pl.BlockSpec((blk, dim), lambda i: (i, 0)),
            pl.BlockSpec((dim, n_exp), lambda i: (0, 0)),
            pl.BlockSpec((1, n_exp), lambda i: (0, 0)),
            pl.BlockSpec((n_exp * dim, dimp), lambda i: (0, 0)),
            pl.BlockSpec((dimp, dim), lambda i: (0, 0)),
            pl.BlockSpec((1, dim), lambda i: (0, 0)),
            pl.BlockSpec((heads * dhp, heads), lambda i: (0, 0)),
            pl.BlockSpec((heads, heads * dhp), lambda i: (0, 0)),
        ],
        out_specs=pl.BlockSpec((blk, dim), lambda i: (i, 0)),
        out_shape=jax.ShapeDtypeStruct((b, dim), jnp.float32),
    )(x, Wg, bg.reshape(1, n_exp), weff, wproj_pad.astype(jnp.bfloat16),
      bproj.reshape(1, dim), bd, kb2)
    return out
